# Initial kernel scaffold; baseline (speedup 1.0000x reference)
import jax
import jax.numpy as jnp
from jax.experimental import pallas as pl


def kernel(q, action_mask):
    raise NotImplementedError("write your pallas kernel here")


# speedup vs baseline: 2.8796x; 2.8796x over previous
import functools

import jax
import jax.numpy as jnp
from jax import lax
from jax.experimental import pallas as pl
from jax.experimental.pallas import tpu as pltpu
from jax.experimental.pallas import tpu_sc as plsc

ACT_DIM = 100000
BATCH = 128
NIDX = 1000
CPAD = 1024
NW = 32
ROWS_PER_W = BATCH // NW
SUBROWS = ROWS_PER_W * (CPAD // 128)
SENTINEL = ACT_DIM


def _sc_gather(qflat, idx_pad):
  mesh = plsc.VectorSubcoreMesh(core_axis_name="c", subcore_axis_name="s")

  @functools.partial(
      pl.kernel,
      out_type=jax.ShapeDtypeStruct((NW, SUBROWS, 128), jnp.float32),
      mesh=mesh,
      scratch_types=[
          pltpu.VMEM((CPAD,), jnp.int32),
          pltpu.VMEM((SUBROWS, 128), jnp.int32),
          pltpu.VMEM((SUBROWS, 128), jnp.float32),
          pltpu.SemaphoreType.DMA,
      ],
  )
  def k(qflat_hbm, idx_hbm, g_hbm, idx_v, flat_v, rows_v, sem):
    wid = lax.axis_index("s") * 2 + lax.axis_index("c")
    base_row = wid * ROWS_PER_W
    pltpu.sync_copy(idx_hbm, idx_v)

    @pl.loop(0, ROWS_PER_W)
    def _(i):
      row_off = (base_row + i) * ACT_DIM

      @pl.loop(0, CPAD // 128)
      def _(j):
        for c in range(8):
          v = idx_v[pl.ds(j * 128 + c * 16, 16)]
          flat_v[i * 8 + j, pl.ds(c * 16, 16)] = (
              jnp.minimum(v, ACT_DIM - 1) + row_off)

    @pl.loop(0, SUBROWS // 8)
    def _(g):
      copies = []
      for b in range(8):
        s = g * 8 + b
        copies.append(
            pltpu.async_copy(qflat_hbm.at[flat_v.at[s]], rows_v.at[s], sem))
      for cp in copies:
        cp.wait()

    pltpu.sync_copy(rows_v, g_hbm.at[wid])

  return k(qflat, idx_pad)


def _tf_rotl(x, d):
  return lax.shift_left(x, jnp.int32(d)) | lax.shift_right_logical(
      x, jnp.int32(32 - d))


def _tf_round4(x0, x1, rots):
  for r in rots:
    x0 = x0 + x1
    x1 = _tf_rotl(x1, r)
    x1 = x0 ^ x1
  return x0, x1


def _gumbel_bits(f):
  ks0 = jnp.int32(0)
  ks1 = jnp.int32(42)
  ks2 = jnp.int32(0x1BD11BDA ^ 42)
  rots_a = (13, 15, 26, 6)
  rots_b = (17, 29, 16, 24)
  x0 = jnp.zeros_like(f) + ks0
  x1 = f + ks1
  x0, x1 = _tf_round4(x0, x1, rots_a)
  x0 = x0 + ks1; x1 = x1 + ks2 + jnp.int32(1)
  x0, x1 = _tf_round4(x0, x1, rots_b)
  x0 = x0 + ks2; x1 = x1 + ks0 + jnp.int32(2)
  x0, x1 = _tf_round4(x0, x1, rots_a)
  x0 = x0 + ks0; x1 = x1 + ks1 + jnp.int32(3)
  x0, x1 = _tf_round4(x0, x1, rots_b)
  x0 = x0 + ks1; x1 = x1 + ks2 + jnp.int32(4)
  x0, x1 = _tf_round4(x0, x1, rots_a)
  x0 = x0 + ks2; x1 = x1 + ks0 + jnp.int32(5)
  return x0 ^ x1


def _gumbel(f):
  bits = _gumbel_bits(f)
  float_bits = lax.shift_right_logical(bits, jnp.int32(9)) | jnp.int32(
      0x3F800000)
  fl = lax.bitcast_convert_type(float_bits, jnp.float32) - jnp.float32(1.0)
  tiny = jnp.float32(1.1754944e-38)
  u = jnp.maximum(tiny, fl * (jnp.float32(1.0) - tiny) + tiny)
  return -jnp.log(-jnp.log(u))


def _tc_body(g_ref, idx_ref, idxc_ref, act_ref, logp_ref):
  g = g_ref[...]
  cidx = idx_ref[...]
  cidx_c = idxc_ref[...]
  pos_i = lax.broadcasted_iota(jnp.int32, (CPAD, CPAD), 0)
  pos_j = lax.broadcasted_iota(jnp.int32, (CPAD, CPAD), 1)
  dup = jnp.any((cidx_c == cidx) & (pos_i < pos_j), axis=0, keepdims=True)
  col = lax.broadcasted_iota(jnp.int32, (1, CPAD), 1)
  valid = jnp.logical_not(dup) & (col < NIDX)
  validb = jnp.broadcast_to(valid, (BATCH, CPAD))
  neg_inf = jnp.float32(-jnp.inf)

  gm = jnp.where(validb, g, neg_inf)
  m = jnp.max(gm, axis=1, keepdims=True)
  shifted = gm - m
  sumexp = jnp.sum(jnp.where(validb, jnp.exp(shifted), jnp.float32(0.0)),
                   axis=1, keepdims=True)
  pi_log = shifted - jnp.log(sumexp)

  r = lax.broadcasted_iota(jnp.int32, (BATCH, CPAD), 0)
  f = r * ACT_DIM + jnp.broadcast_to(cidx, (BATCH, CPAD))
  z = jnp.where(validb, pi_log + _gumbel(f), neg_inf)
  zmax = jnp.max(z, axis=1, keepdims=True)
  is_max = (z == zmax) & validb
  cidx_b = jnp.broadcast_to(cidx, (BATCH, CPAD))
  win_c = jnp.min(jnp.where(is_max, cidx_b, jnp.int32(2**31 - 1)),
                  axis=1, keepdims=True)
  sel = is_max & (cidx_b == win_c)
  logp = jnp.max(jnp.where(sel, pi_log, neg_inf), axis=1, keepdims=True)
  act_ref[...] = win_c
  logp_ref[...] = logp


def _tc_sample(g, idx2d, idx2d_col, interpret=False):
  return pl.pallas_call(
      _tc_body,
      out_shape=(
          jax.ShapeDtypeStruct((BATCH, 1), jnp.int32),
          jax.ShapeDtypeStruct((BATCH, 1), jnp.float32),
      ),
      interpret=interpret,
  )(g, idx2d, idx2d_col)


def kernel(q, action_mask):
  idx = action_mask.astype(jnp.int32)
  idx_pad = jnp.concatenate(
      [idx, jnp.full((CPAD - NIDX,), SENTINEL, jnp.int32)])
  g3 = _sc_gather(q.reshape(-1), idx_pad)
  g = g3.reshape(BATCH, CPAD)
  pi_action, logp_pi = _tc_sample(
      g, idx_pad.reshape(1, CPAD), idx_pad.reshape(CPAD, 1))
  return pi_action, logp_pi


# speedup vs baseline: 4.5090x; 1.5658x over previous
import functools

import jax
import jax.numpy as jnp
from jax import lax
from jax.experimental import pallas as pl
from jax.experimental.pallas import tpu as pltpu
from jax.experimental.pallas import tpu_sc as plsc

ACT_DIM = 100000
BATCH = 128
NIDX = 1000
CPAD = 1024
NW = 32
ROWS_PER_W = BATCH // NW
SUBROWS = ROWS_PER_W * (CPAD // 128)
SENTINEL = ACT_DIM


def _sc_gather(q, idx_pad):
  mesh = plsc.VectorSubcoreMesh(core_axis_name="c", subcore_axis_name="s")

  @functools.partial(
      pl.kernel,
      out_type=jax.ShapeDtypeStruct((BATCH, CPAD), jnp.float32),
      mesh=mesh,
      scratch_types=[
          pltpu.VMEM((CPAD,), jnp.int32),
          pltpu.VMEM((ACT_DIM,), jnp.float32),
          pltpu.VMEM((ROWS_PER_W, CPAD), jnp.float32),
          pltpu.SemaphoreType.DMA,
      ],
      compiler_params=pltpu.CompilerParams(needs_layout_passes=False),
  )
  def k(q_hbm, idx_hbm, g_hbm, idx_v, row_v, out_v, sem):
    wid = lax.axis_index("s") * 2 + lax.axis_index("c")
    base_row = wid * ROWS_PER_W
    pltpu.sync_copy(idx_hbm, idx_v)

    @pl.loop(0, CPAD // 16)
    def _(t):
      idx_v[pl.ds(t * 16, 16)] = jnp.minimum(
          idx_v[pl.ds(t * 16, 16)], ACT_DIM - 1)

    @pl.loop(0, ROWS_PER_W)
    def _(i):
      pltpu.sync_copy(q_hbm.at[base_row + i], row_v)

      @pl.loop(0, CPAD // 16)
      def _(t):
        cv = idx_v[pl.ds(t * 16, 16)]
        out_v[i, pl.ds(t * 16, 16)] = plsc.load_gather(row_v, [cv])

    pltpu.sync_copy(out_v, g_hbm.at[pl.ds(base_row, ROWS_PER_W)])

  return k(q, idx_pad)


def _tf_rotl(x, d):
  return lax.shift_left(x, jnp.int32(d)) | lax.shift_right_logical(
      x, jnp.int32(32 - d))


def _tf_round4(x0, x1, rots):
  for r in rots:
    x0 = x0 + x1
    x1 = _tf_rotl(x1, r)
    x1 = x0 ^ x1
  return x0, x1


def _gumbel_bits(f):
  ks0 = jnp.int32(0)
  ks1 = jnp.int32(42)
  ks2 = jnp.int32(0x1BD11BDA ^ 42)
  rots_a = (13, 15, 26, 6)
  rots_b = (17, 29, 16, 24)
  x0 = jnp.zeros_like(f) + ks0
  x1 = f + ks1
  x0, x1 = _tf_round4(x0, x1, rots_a)
  x0 = x0 + ks1; x1 = x1 + ks2 + jnp.int32(1)
  x0, x1 = _tf_round4(x0, x1, rots_b)
  x0 = x0 + ks2; x1 = x1 + ks0 + jnp.int32(2)
  x0, x1 = _tf_round4(x0, x1, rots_a)
  x0 = x0 + ks0; x1 = x1 + ks1 + jnp.int32(3)
  x0, x1 = _tf_round4(x0, x1, rots_b)
  x0 = x0 + ks1; x1 = x1 + ks2 + jnp.int32(4)
  x0, x1 = _tf_round4(x0, x1, rots_a)
  x0 = x0 + ks2; x1 = x1 + ks0 + jnp.int32(5)
  return x0 ^ x1


def _gumbel(f):
  bits = _gumbel_bits(f)
  float_bits = lax.shift_right_logical(bits, jnp.int32(9)) | jnp.int32(
      0x3F800000)
  fl = lax.bitcast_convert_type(float_bits, jnp.float32) - jnp.float32(1.0)
  tiny = jnp.float32(1.1754944e-38)
  u = jnp.maximum(tiny, fl * (jnp.float32(1.0) - tiny) + tiny)
  return -jnp.log(-jnp.log(u))


def _tc_body(g_ref, idx_ref, idxc_ref, act_ref, logp_ref):
  g = g_ref[...]
  cidx = idx_ref[...]
  cidx_c = idxc_ref[...]
  pos_i = lax.broadcasted_iota(jnp.int32, (CPAD, CPAD), 0)
  pos_j = lax.broadcasted_iota(jnp.int32, (CPAD, CPAD), 1)
  dup = jnp.any((cidx_c == cidx) & (pos_i < pos_j), axis=0, keepdims=True)
  col = lax.broadcasted_iota(jnp.int32, (1, CPAD), 1)
  valid = jnp.logical_not(dup) & (col < NIDX)
  validb = jnp.broadcast_to(valid, (BATCH, CPAD))
  neg_inf = jnp.float32(-jnp.inf)

  gm = jnp.where(validb, g, neg_inf)
  m = jnp.max(gm, axis=1, keepdims=True)
  shifted = gm - m
  sumexp = jnp.sum(jnp.where(validb, jnp.exp(shifted), jnp.float32(0.0)),
                   axis=1, keepdims=True)
  pi_log = shifted - jnp.log(sumexp)

  r = lax.broadcasted_iota(jnp.int32, (BATCH, CPAD), 0)
  f = r * ACT_DIM + jnp.broadcast_to(cidx, (BATCH, CPAD))
  z = jnp.where(validb, pi_log + _gumbel(f), neg_inf)
  zmax = jnp.max(z, axis=1, keepdims=True)
  is_max = (z == zmax) & validb
  cidx_b = jnp.broadcast_to(cidx, (BATCH, CPAD))
  win_c = jnp.min(jnp.where(is_max, cidx_b, jnp.int32(2**31 - 1)),
                  axis=1, keepdims=True)
  sel = is_max & (cidx_b == win_c)
  logp = jnp.max(jnp.where(sel, pi_log, neg_inf), axis=1, keepdims=True)
  act_ref[...] = win_c
  logp_ref[...] = logp


def _tc_sample(g, idx2d, idx2d_col, interpret=False):
  return pl.pallas_call(
      _tc_body,
      out_shape=(
          jax.ShapeDtypeStruct((BATCH, 1), jnp.int32),
          jax.ShapeDtypeStruct((BATCH, 1), jnp.float32),
      ),
      interpret=interpret,
  )(g, idx2d, idx2d_col)


def kernel(q, action_mask):
  idx = action_mask.astype(jnp.int32)
  idx_pad = jnp.concatenate(
      [idx, jnp.full((CPAD - NIDX,), SENTINEL, jnp.int32)])
  g = _sc_gather(q, idx_pad)
  pi_action, logp_pi = _tc_sample(
      g, idx_pad.reshape(1, CPAD), idx_pad.reshape(CPAD, 1))
  return pi_action, logp_pi


# speedup vs baseline: 4.5298x; 1.0046x over previous
import functools

import jax
import jax.numpy as jnp
from jax import lax
from jax.experimental import pallas as pl
from jax.experimental.pallas import tpu as pltpu
from jax.experimental.pallas import tpu_sc as plsc

ACT_DIM = 100000
BATCH = 128
NIDX = 1000
CPAD = 1024
NW = 32
ROWS_PER_W = BATCH // NW
SUBROWS = ROWS_PER_W * (CPAD // 128)
SENTINEL = ACT_DIM


def _sc_gather(q, idx_pad):
  mesh = plsc.VectorSubcoreMesh(core_axis_name="c", subcore_axis_name="s")

  @functools.partial(
      pl.kernel,
      out_type=jax.ShapeDtypeStruct((BATCH, CPAD), jnp.float32),
      mesh=mesh,
      scratch_types=[
          pltpu.VMEM((CPAD,), jnp.int32),
          pltpu.VMEM((ACT_DIM,), jnp.float32),
          pltpu.VMEM((ROWS_PER_W, CPAD), jnp.float32),
          pltpu.SemaphoreType.DMA,
      ],
      compiler_params=pltpu.CompilerParams(
          needs_layout_passes=False,
          use_tc_tiling_on_sc=True,
      ),
  )
  def k(q_hbm, idx_hbm, g_hbm, idx_v, row_v, out_v, sem):
    wid = lax.axis_index("s") * 2 + lax.axis_index("c")
    base_row = wid * ROWS_PER_W
    pltpu.sync_copy(idx_hbm, idx_v)

    @pl.loop(0, CPAD // 16)
    def _(t):
      idx_v[pl.ds(t * 16, 16)] = jnp.minimum(
          idx_v[pl.ds(t * 16, 16)], ACT_DIM - 1)

    @pl.loop(0, ROWS_PER_W)
    def _(i):
      pltpu.sync_copy(q_hbm.at[base_row + i], row_v)

      @pl.loop(0, CPAD // 16)
      def _(t):
        cv = idx_v[pl.ds(t * 16, 16)]
        out_v[i, pl.ds(t * 16, 16)] = plsc.load_gather(row_v, [cv])

    pltpu.sync_copy(out_v, g_hbm.at[pl.ds(base_row, ROWS_PER_W)])

  return k(q, idx_pad)


def _tf_rotl(x, d):
  return lax.shift_left(x, jnp.int32(d)) | lax.shift_right_logical(
      x, jnp.int32(32 - d))


def _tf_round4(x0, x1, rots):
  for r in rots:
    x0 = x0 + x1
    x1 = _tf_rotl(x1, r)
    x1 = x0 ^ x1
  return x0, x1


def _gumbel_bits(f):
  ks0 = jnp.int32(0)
  ks1 = jnp.int32(42)
  ks2 = jnp.int32(0x1BD11BDA ^ 42)
  rots_a = (13, 15, 26, 6)
  rots_b = (17, 29, 16, 24)
  x0 = jnp.zeros_like(f) + ks0
  x1 = f + ks1
  x0, x1 = _tf_round4(x0, x1, rots_a)
  x0 = x0 + ks1; x1 = x1 + ks2 + jnp.int32(1)
  x0, x1 = _tf_round4(x0, x1, rots_b)
  x0 = x0 + ks2; x1 = x1 + ks0 + jnp.int32(2)
  x0, x1 = _tf_round4(x0, x1, rots_a)
  x0 = x0 + ks0; x1 = x1 + ks1 + jnp.int32(3)
  x0, x1 = _tf_round4(x0, x1, rots_b)
  x0 = x0 + ks1; x1 = x1 + ks2 + jnp.int32(4)
  x0, x1 = _tf_round4(x0, x1, rots_a)
  x0 = x0 + ks2; x1 = x1 + ks0 + jnp.int32(5)
  return x0 ^ x1


def _gumbel(f):
  bits = _gumbel_bits(f)
  float_bits = lax.shift_right_logical(bits, jnp.int32(9)) | jnp.int32(
      0x3F800000)
  fl = lax.bitcast_convert_type(float_bits, jnp.float32) - jnp.float32(1.0)
  tiny = jnp.float32(1.1754944e-38)
  u = jnp.maximum(tiny, fl * (jnp.float32(1.0) - tiny) + tiny)
  return -jnp.log(-jnp.log(u))


def _tc_body(g_ref, idx_ref, idxc_ref, act_ref, logp_ref):
  g = g_ref[...]
  cidx = idx_ref[...]
  cidx_c = idxc_ref[...]
  pos_i = lax.broadcasted_iota(jnp.int32, (CPAD, CPAD), 0)
  pos_j = lax.broadcasted_iota(jnp.int32, (CPAD, CPAD), 1)
  dup = jnp.any((cidx_c == cidx) & (pos_i < pos_j), axis=0, keepdims=True)
  col = lax.broadcasted_iota(jnp.int32, (1, CPAD), 1)
  valid = jnp.logical_not(dup) & (col < NIDX)
  validb = jnp.broadcast_to(valid, (BATCH, CPAD))
  neg_inf = jnp.float32(-jnp.inf)

  gm = jnp.where(validb, g, neg_inf)
  m = jnp.max(gm, axis=1, keepdims=True)
  shifted = gm - m
  sumexp = jnp.sum(jnp.where(validb, jnp.exp(shifted), jnp.float32(0.0)),
                   axis=1, keepdims=True)
  pi_log = shifted - jnp.log(sumexp)

  r = lax.broadcasted_iota(jnp.int32, (BATCH, CPAD), 0)
  f = r * ACT_DIM + jnp.broadcast_to(cidx, (BATCH, CPAD))
  z = jnp.where(validb, pi_log + _gumbel(f), neg_inf)
  zmax = jnp.max(z, axis=1, keepdims=True)
  is_max = (z == zmax) & validb
  cidx_b = jnp.broadcast_to(cidx, (BATCH, CPAD))
  win_c = jnp.min(jnp.where(is_max, cidx_b, jnp.int32(2**31 - 1)),
                  axis=1, keepdims=True)
  sel = is_max & (cidx_b == win_c)
  logp = jnp.max(jnp.where(sel, pi_log, neg_inf), axis=1, keepdims=True)
  act_ref[...] = win_c
  logp_ref[...] = logp


def _tc_sample(g, idx2d, idx2d_col, interpret=False):
  return pl.pallas_call(
      _tc_body,
      out_shape=(
          jax.ShapeDtypeStruct((BATCH, 1), jnp.int32),
          jax.ShapeDtypeStruct((BATCH, 1), jnp.float32),
      ),
      interpret=interpret,
  )(g, idx2d, idx2d_col)


def kernel(q, action_mask):
  idx = action_mask.astype(jnp.int32)
  idx_pad = jnp.concatenate(
      [idx, jnp.full((CPAD - NIDX,), SENTINEL, jnp.int32)])
  g = _sc_gather(q, idx_pad)
  pi_action, logp_pi = _tc_sample(
      g, idx_pad.reshape(1, CPAD), idx_pad.reshape(CPAD, 1))
  return pi_action, logp_pi


# speedup vs baseline: 17.0537x; 3.7648x over previous
import functools

import jax
import jax.numpy as jnp
from jax import lax
from jax.experimental import pallas as pl
from jax.experimental.pallas import tpu as pltpu
from jax.experimental.pallas import tpu_sc as plsc

ACT_DIM = 100000
BATCH = 128
NIDX = 1000
CPAD = 1024
NW = 32
SLOTS_PER_W = CPAD // NW
SENTINEL = ACT_DIM


def _sc_gather(qt, idx):
  mesh = plsc.VectorSubcoreMesh(core_axis_name="c", subcore_axis_name="s")

  @functools.partial(
      pl.kernel,
      out_type=jax.ShapeDtypeStruct((CPAD, BATCH), jnp.float32),
      mesh=mesh,
      scratch_types=[
          pltpu.VMEM((SLOTS_PER_W,), jnp.int32),
          pltpu.VMEM((SLOTS_PER_W, BATCH), jnp.float32),
          pltpu.SemaphoreType.DMA,
      ],
      compiler_params=pltpu.CompilerParams(
          needs_layout_passes=False,
          use_tc_tiling_on_sc=True,
          skip_device_barrier=True,
      ),
  )
  def k(qt_hbm, idx_hbm, gt_hbm, idx_v, rows_v, sem):
    wid = lax.axis_index("s") * 2 + lax.axis_index("c")
    base = wid * SLOTS_PER_W

    @pl.when(wid < NW - 1)
    def _():
      pltpu.sync_copy(idx_hbm.at[pl.ds(base, SLOTS_PER_W)], idx_v)

    @pl.when(wid == NW - 1)
    def _():
      ntail = NIDX - (NW - 1) * SLOTS_PER_W
      pltpu.sync_copy(idx_hbm.at[pl.ds(NIDX - ntail, ntail)],
                      idx_v.at[pl.ds(0, ntail)])
      lane = lax.iota(jnp.int32, 16)
      head = idx_v[pl.ds(0, 16)]
      idx_v[pl.ds(0, 16)] = jnp.where(
          lane < ntail, head, jnp.int32(ACT_DIM - 1))
      idx_v[pl.ds(16, 16)] = jnp.full((16,), ACT_DIM - 1, jnp.int32)

    pltpu.async_copy(qt_hbm.at[idx_v], rows_v, sem).wait()
    pltpu.sync_copy(rows_v, gt_hbm.at[pl.ds(base, SLOTS_PER_W)])

  return k(qt, idx)


def _tf_rotl(x, d):
  return lax.shift_left(x, jnp.int32(d)) | lax.shift_right_logical(
      x, jnp.int32(32 - d))


def _tf_round4(x0, x1, rots):
  for r in rots:
    x0 = x0 + x1
    x1 = _tf_rotl(x1, r)
    x1 = x0 ^ x1
  return x0, x1


def _gumbel_bits(f):
  ks0 = jnp.int32(0)
  ks1 = jnp.int32(42)
  ks2 = jnp.int32(0x1BD11BDA ^ 42)
  rots_a = (13, 15, 26, 6)
  rots_b = (17, 29, 16, 24)
  x0 = jnp.zeros_like(f) + ks0
  x1 = f + ks1
  x0, x1 = _tf_round4(x0, x1, rots_a)
  x0 = x0 + ks1; x1 = x1 + ks2 + jnp.int32(1)
  x0, x1 = _tf_round4(x0, x1, rots_b)
  x0 = x0 + ks2; x1 = x1 + ks0 + jnp.int32(2)
  x0, x1 = _tf_round4(x0, x1, rots_a)
  x0 = x0 + ks0; x1 = x1 + ks1 + jnp.int32(3)
  x0, x1 = _tf_round4(x0, x1, rots_b)
  x0 = x0 + ks1; x1 = x1 + ks2 + jnp.int32(4)
  x0, x1 = _tf_round4(x0, x1, rots_a)
  x0 = x0 + ks2; x1 = x1 + ks0 + jnp.int32(5)
  return x0 ^ x1


def _gumbel(f):
  bits = _gumbel_bits(f)
  float_bits = lax.shift_right_logical(bits, jnp.int32(9)) | jnp.int32(
      0x3F800000)
  fl = lax.bitcast_convert_type(float_bits, jnp.float32) - jnp.float32(1.0)
  tiny = jnp.float32(1.1754944e-38)
  u = jnp.maximum(tiny, fl * (jnp.float32(1.0) - tiny) + tiny)
  return -jnp.log(-jnp.log(u))


def _tc_noise_body(idx_ref, idxc_ref, noise_ref, valid_ref):
  cidx = idx_ref[...]
  cidx_c = idxc_ref[...]
  pos_a = lax.broadcasted_iota(jnp.int32, (CPAD, CPAD), 0)
  pos_b = lax.broadcasted_iota(jnp.int32, (CPAD, CPAD), 1)
  dup = jnp.any((cidx_c == cidx) & (pos_b < pos_a), axis=1, keepdims=True)
  slot = lax.broadcasted_iota(jnp.int32, (CPAD, 1), 0)
  valid = jnp.logical_not(dup) & (slot < NIDX)
  validb = jnp.broadcast_to(valid, (CPAD, BATCH))

  r = lax.broadcasted_iota(jnp.int32, (CPAD, BATCH), 1)
  f = r * ACT_DIM + jnp.broadcast_to(cidx_c, (CPAD, BATCH))
  noise_ref[...] = jnp.where(validb, _gumbel(f), jnp.float32(-jnp.inf))
  valid_ref[...] = valid.astype(jnp.int32)


def _tc_combine_body(gt_ref, noise_ref, valid_ref, idxc_ref, act_ref,
                     logp_ref):
  gt = gt_ref[...]
  noise = noise_ref[...]
  valid = valid_ref[...] == 1
  cidx_c = idxc_ref[...]
  validb = jnp.broadcast_to(valid, (CPAD, BATCH))
  neg_inf = jnp.float32(-jnp.inf)

  gm = jnp.where(validb, gt, neg_inf)
  m = jnp.max(gm, axis=0, keepdims=True)
  shifted = gm - m
  sumexp = jnp.sum(jnp.where(validb, jnp.exp(shifted), jnp.float32(0.0)),
                   axis=0, keepdims=True)
  pi_log = shifted - jnp.log(sumexp)

  z = jnp.where(validb, pi_log + noise, neg_inf)
  zmax = jnp.max(z, axis=0, keepdims=True)
  is_max = (z == zmax) & validb
  cidx_b = jnp.broadcast_to(cidx_c, (CPAD, BATCH))
  win_c = jnp.min(jnp.where(is_max, cidx_b, jnp.int32(2**31 - 1)),
                  axis=0, keepdims=True)
  sel = is_max & (cidx_b == win_c)
  logp = jnp.max(jnp.where(sel, pi_log, neg_inf), axis=0, keepdims=True)
  act_ref[...] = win_c
  logp_ref[...] = logp


def _tc_sample(gt, idx2d, idx2d_col, interpret=False):
  params = None if interpret else pltpu.CompilerParams(
      skip_device_barrier=True)
  noise, valid = pl.pallas_call(
      _tc_noise_body,
      out_shape=(
          jax.ShapeDtypeStruct((CPAD, BATCH), jnp.float32),
          jax.ShapeDtypeStruct((CPAD, 1), jnp.int32),
      ),
      interpret=interpret,
      compiler_params=params,
  )(idx2d, idx2d_col)
  return pl.pallas_call(
      _tc_combine_body,
      out_shape=(
          jax.ShapeDtypeStruct((1, BATCH), jnp.int32),
          jax.ShapeDtypeStruct((1, BATCH), jnp.float32),
      ),
      interpret=interpret,
      compiler_params=params,
  )(gt, noise, valid, idx2d_col)


def kernel(q, action_mask):
  idx = action_mask.astype(jnp.int32)
  gt = _sc_gather(q.T, idx)
  idx_pad = jnp.concatenate(
      [idx, jnp.full((CPAD - NIDX,), SENTINEL, jnp.int32)])
  act, logp = _tc_sample(gt, idx_pad.reshape(1, CPAD), idx_pad.reshape(CPAD, 1))
  return act.reshape(BATCH, 1), logp.reshape(BATCH, 1)
